# baseline (speedup 1.0000x reference)
import jax
import jax.numpy as jnp
from jax import lax
from jax.experimental import pallas as pl
from jax.experimental.pallas import tpu as pltpu
from jax.experimental.pallas import tpu_sc as plsc

BATCH = 64
HW = 32
D = 256
NC = 2
NS = 16
WINDOW = 8


def _sc_body(row_hbm, col_hbm, out_hbm, chunk, sem):
    k = lax.axis_index("c") * NS + lax.axis_index("s")
    pltpu.sync_copy(col_hbm, chunk.at[:, pl.ds(0, D)])
    for i in range(HW):
        pltpu.sync_copy(row_hbm.at[pl.ds(k, 1)], chunk.at[pl.ds(i, 1), pl.ds(D, D)])
    copies = [
        pltpu.make_async_copy(
            chunk, out_hbm.at[pl.ds(b * (HW * HW) + k * HW, HW)], sem
        )
        for b in range(BATCH)
    ]
    for b in range(BATCH):
        copies[b].start()
        if b >= WINDOW:
            copies[b - WINDOW].wait()
    for b in range(BATCH - WINDOW, BATCH):
        copies[b].wait()


def kernel(row_embeddings, column_embeddings):
    row = row_embeddings[:HW]
    col = column_embeddings[:HW]

    mesh = plsc.VectorSubcoreMesh(core_axis_name="c", subcore_axis_name="s")
    sc_kernel = pl.kernel(
        _sc_body,
        out_type=jax.ShapeDtypeStruct((BATCH * HW * HW, 2 * D), jnp.float32),
        mesh=mesh,
        scratch_types=[
            pltpu.VMEM((HW, 2 * D), jnp.float32),
            pltpu.SemaphoreType.DMA,
        ],
    )
    out = sc_kernel(row, col)
    return out.reshape(BATCH, HW * HW, 2 * D)
